# baseline (speedup 1.0000x reference)
import functools

import jax
import jax.numpy as jnp
from jax import lax
from jax.experimental import pallas as pl
from jax.experimental.pallas import tpu as pltpu
from jax.experimental.pallas import tpu_sc as plsc

B, N, K = 2, 5000, 16
D_IN, D_COND, D_OUT = 256, 128, 256
NODES = B * N
NW = 32
C = 320
PAD_NODES = NW * C
SUB = 16
NSUB = C // SUB
ROWS_BLK = 2000
LANES = 16


def _dense_body(nf_ref, cond_ref, wc_ref, bc_ref, wf_ref, bf_ref, x_ref):
    j = pl.program_id(0)
    x = jnp.dot(nf_ref[...], wf_ref[...], preferred_element_type=jnp.float32)
    x = x + bf_ref[...]
    mu = jnp.mean(x, axis=-1, keepdims=True)
    var = jnp.mean((x - mu) ** 2, axis=-1, keepdims=True)
    x = (x - mu) * lax.rsqrt(var + 1e-5)
    gb = jnp.dot(cond_ref[...], wc_ref[...], preferred_element_type=jnp.float32)
    gb = gb + bc_ref[...]
    gamma = gb[:, :D_OUT] + 1.0
    beta = gb[:, D_OUT:]
    rows = j * ROWS_BLK + lax.broadcasted_iota(jnp.int32, (ROWS_BLK, 1), 0)
    in_b1 = rows >= N
    g = jnp.where(in_b1, gamma[1:2, :], gamma[0:1, :])
    bt = jnp.where(in_b1, beta[1:2, :], beta[0:1, :])
    x_ref[...] = jnp.maximum(x * g + bt, 0.0)


def _dense(nf, cond2d, Wc, bc2d, Wf, bf2d):
    return pl.pallas_call(
        _dense_body,
        grid=(NODES // ROWS_BLK,),
        in_specs=[
            pl.BlockSpec((ROWS_BLK, D_IN), lambda j: (j, 0)),
            pl.BlockSpec((B, D_COND), lambda j: (0, 0)),
            pl.BlockSpec((D_COND, 2 * D_OUT), lambda j: (0, 0)),
            pl.BlockSpec((1, 2 * D_OUT), lambda j: (0, 0)),
            pl.BlockSpec((D_IN, D_OUT), lambda j: (0, 0)),
            pl.BlockSpec((1, D_OUT), lambda j: (0, 0)),
        ],
        out_specs=pl.BlockSpec((ROWS_BLK, D_OUT), lambda j: (j, 0)),
        out_shape=jax.ShapeDtypeStruct((NODES, D_OUT), jnp.float32),
    )(nf, cond2d, Wc, bc2d, Wf, bf2d)


def _sc_body(x_hbm, coords_hbm, ew_hbm, ep_hbm, out_hbm,
             coords_v, ew_v, ep_v, xres_v, out_v, rows_v, sem):
    cid = lax.axis_index("c")
    sid = lax.axis_index("s")
    wid = sid * 2 + cid
    base_node = wid * C
    base_edge = base_node * K

    pltpu.sync_copy(coords_hbm.at[pl.ds(base_edge, C * K)], coords_v)
    pltpu.sync_copy(ew_hbm.at[pl.ds(base_edge, C * K)], ew_v)
    pltpu.sync_copy(ep_hbm.at[pl.ds(base_edge, C * K)], ep_v)

    def sub_body(s, carry):
        node0 = base_node + s * SUB
        row0 = jnp.minimum(node0, NODES - SUB)
        pltpu.sync_copy(x_hbm.at[pl.ds(row0, SUB)], xres_v)

        def node_body(i, carry2):
            e0 = (s * SUB + i) * K
            pltpu.async_copy(
                x_hbm.at[coords_v.at[pl.ds(e0, K)]], rows_v, sem).wait()
            wv = ew_v[pl.ds(e0, LANES)] * ep_v[pl.ds(e0, LANES)]
            accs = [xres_v[i, pl.ds(d * LANES, LANES)] for d in range(D_OUT // LANES)]
            for k in range(K):
                wk = wv.at[jnp.full((LANES,), k, jnp.int32)].get(
                    mode="promise_in_bounds")
                for d in range(D_OUT // LANES):
                    accs[d] = accs[d] + wk * rows_v[k, pl.ds(d * LANES, LANES)]
            for d in range(D_OUT // LANES):
                out_v[i, pl.ds(d * LANES, LANES)] = jnp.maximum(accs[d], 0.0)
            return carry2

        lax.fori_loop(0, SUB, node_body, 0)
        pltpu.sync_copy(out_v, out_hbm.at[pl.ds(node0, SUB)])
        return carry

    lax.fori_loop(0, NSUB, sub_body, 0)


_sc_msg = functools.partial(
    pl.kernel,
    mesh=plsc.VectorSubcoreMesh(core_axis_name="c", subcore_axis_name="s"),
    out_type=jax.ShapeDtypeStruct((PAD_NODES, D_OUT), jnp.float32),
    scratch_types=[
        pltpu.VMEM((C * K,), jnp.int32),
        pltpu.VMEM((C * K,), jnp.float32),
        pltpu.VMEM((C * K,), jnp.float32),
        pltpu.VMEM((SUB, D_OUT), jnp.float32),
        pltpu.VMEM((SUB, D_OUT), jnp.float32),
        pltpu.VMEM((K, D_OUT), jnp.float32),
        pltpu.SemaphoreType.DMA,
    ],
)(_sc_body)


def kernel(node_feats, cond_feats, edge_weights, edge_params, coords1, Wc, bc, Wf, bf):
    nf = node_feats.reshape(NODES, D_IN)
    cond2d = cond_feats.reshape(B, D_COND)
    x = _dense(nf, cond2d, Wc, bc.reshape(1, -1), Wf, bf.reshape(1, -1))

    pad_e = (PAD_NODES - NODES) * K
    coords = jnp.pad(coords1.astype(jnp.int32), (0, pad_e))
    ew = jnp.pad(edge_weights.reshape(-1), (0, pad_e))
    ep = jnp.pad(edge_params.reshape(-1), (0, pad_e))

    out = _sc_msg(x, coords, ew, ep)
    return out[:NODES].reshape(B, N, D_OUT)


# speedup vs baseline: 1.6790x; 1.6790x over previous
import functools

import jax
import jax.numpy as jnp
from jax import lax
from jax.experimental import pallas as pl
from jax.experimental.pallas import tpu as pltpu
from jax.experimental.pallas import tpu_sc as plsc

B, N, K = 2, 5000, 16
D_IN, D_COND, D_OUT = 256, 128, 256
NODES = B * N
NW = 32
C = 320
PAD_NODES = NW * C
SUB = 8
NSUB = C // SUB
ROWS_BLK = 2000
LANES = 16


def _dense_body(nf_ref, cond_ref, wc_ref, bc_ref, wf_ref, bf_ref, x_ref):
    j = pl.program_id(0)
    x = jnp.dot(nf_ref[...], wf_ref[...], preferred_element_type=jnp.float32)
    x = x + bf_ref[...]
    mu = jnp.mean(x, axis=-1, keepdims=True)
    var = jnp.mean((x - mu) ** 2, axis=-1, keepdims=True)
    x = (x - mu) * lax.rsqrt(var + 1e-5)
    gb = jnp.dot(cond_ref[...], wc_ref[...], preferred_element_type=jnp.float32)
    gb = gb + bc_ref[...]
    gamma = gb[:, :D_OUT] + 1.0
    beta = gb[:, D_OUT:]
    rows = j * ROWS_BLK + lax.broadcasted_iota(jnp.int32, (ROWS_BLK, 1), 0)
    in_b1 = rows >= N
    g = jnp.where(in_b1, gamma[1:2, :], gamma[0:1, :])
    bt = jnp.where(in_b1, beta[1:2, :], beta[0:1, :])
    x_ref[...] = jnp.maximum(x * g + bt, 0.0)


def _dense(nf, cond2d, Wc, bc2d, Wf, bf2d):
    return pl.pallas_call(
        _dense_body,
        grid=(NODES // ROWS_BLK,),
        in_specs=[
            pl.BlockSpec((ROWS_BLK, D_IN), lambda j: (j, 0)),
            pl.BlockSpec((B, D_COND), lambda j: (0, 0)),
            pl.BlockSpec((D_COND, 2 * D_OUT), lambda j: (0, 0)),
            pl.BlockSpec((1, 2 * D_OUT), lambda j: (0, 0)),
            pl.BlockSpec((D_IN, D_OUT), lambda j: (0, 0)),
            pl.BlockSpec((1, D_OUT), lambda j: (0, 0)),
        ],
        out_specs=pl.BlockSpec((ROWS_BLK, D_OUT), lambda j: (j, 0)),
        out_shape=jax.ShapeDtypeStruct((NODES, D_OUT), jnp.float32),
    )(nf, cond2d, Wc, bc2d, Wf, bf2d)


def _sc_body(x_hbm, coords_hbm, ew_hbm, ep_hbm, out_hbm,
             coords_v, ew_v, ep_v, xres0, xres1, out_v,
             rows0, rows1, semr0, semr1, semx0, semx1):
    cid = lax.axis_index("c")
    sid = lax.axis_index("s")
    wid = sid * 2 + cid
    base_node = wid * C
    base_edge = base_node * K

    pltpu.sync_copy(coords_hbm.at[pl.ds(base_edge, C * K)], coords_v)
    pltpu.sync_copy(ew_hbm.at[pl.ds(base_edge, C * K)], ew_v)
    pltpu.sync_copy(ep_hbm.at[pl.ds(base_edge, C * K)], ep_v)

    rows = (rows0, rows1)
    xres = (xres0, xres1)
    semr = (semr0, semr1)
    semx = (semx0, semx1)

    def issue(s, b):
        e0 = s * SUB * K
        pltpu.async_copy(
            x_hbm.at[coords_v.at[pl.ds(e0, SUB * K)]], rows[b], semr[b])
        node0 = base_node + s * SUB
        row0 = jnp.minimum(node0, NODES - SUB)
        pltpu.async_copy(x_hbm.at[pl.ds(row0, SUB)], xres[b], semx[b])

    def drain(b):
        pltpu.make_async_copy(x_hbm.at[pl.ds(0, SUB * K)], rows[b], semr[b]).wait()

    def compute(s, b):
        def node_body(i, carry):
            e0 = (s * SUB + i) * K
            wv = ew_v[pl.ds(e0, LANES)] * ep_v[pl.ds(e0, LANES)]
            accs = [jnp.zeros((LANES,), jnp.float32)] * (D_OUT // LANES)
            for k in range(K):
                wk = wv.at[jnp.full((LANES,), k, jnp.int32)].get(
                    mode="promise_in_bounds")
                r = i * K + k
                for d in range(D_OUT // LANES):
                    accs[d] = accs[d] + wk * rows[b][r, pl.ds(d * LANES, LANES)]
            for d in range(D_OUT // LANES):
                acc = accs[d] + xres[b][i, pl.ds(d * LANES, LANES)]
                out_v[i, pl.ds(d * LANES, LANES)] = jnp.maximum(acc, 0.0)
            return carry

        pltpu.make_async_copy(x_hbm.at[pl.ds(0, SUB)], xres[b], semx[b]).wait()
        lax.fori_loop(0, SUB, node_body, 0)
        pltpu.sync_copy(out_v, out_hbm.at[pl.ds(base_node + s * SUB, SUB)])

    issue(0, 0)

    def outer(g, carry):
        s0 = 2 * g
        s1 = 2 * g + 1
        issue(s1, 1)
        drain(0)
        compute(s0, 0)
        issue(jnp.minimum(s1 + 1, NSUB - 1), 0)
        drain(1)
        compute(s1, 1)
        return carry

    lax.fori_loop(0, NSUB // 2, outer, 0)
    drain(0)
    pltpu.make_async_copy(x_hbm.at[pl.ds(0, SUB)], xres[0], semx[0]).wait()


_sc_msg = functools.partial(
    pl.kernel,
    mesh=plsc.VectorSubcoreMesh(core_axis_name="c", subcore_axis_name="s"),
    out_type=jax.ShapeDtypeStruct((PAD_NODES, D_OUT), jnp.float32),
    scratch_types=[
        pltpu.VMEM((C * K,), jnp.int32),
        pltpu.VMEM((C * K,), jnp.float32),
        pltpu.VMEM((C * K,), jnp.float32),
        pltpu.VMEM((SUB, D_OUT), jnp.float32),
        pltpu.VMEM((SUB, D_OUT), jnp.float32),
        pltpu.VMEM((SUB, D_OUT), jnp.float32),
        pltpu.VMEM((SUB * K, D_OUT), jnp.float32),
        pltpu.VMEM((SUB * K, D_OUT), jnp.float32),
        pltpu.SemaphoreType.DMA,
        pltpu.SemaphoreType.DMA,
        pltpu.SemaphoreType.DMA,
        pltpu.SemaphoreType.DMA,
    ],
)(_sc_body)


def kernel(node_feats, cond_feats, edge_weights, edge_params, coords1, Wc, bc, Wf, bf):
    nf = node_feats.reshape(NODES, D_IN)
    cond2d = cond_feats.reshape(B, D_COND)
    x = _dense(nf, cond2d, Wc, bc.reshape(1, -1), Wf, bf.reshape(1, -1))

    pad_e = (PAD_NODES - NODES) * K
    coords = jnp.pad(coords1.astype(jnp.int32), (0, pad_e))
    ew = jnp.pad(edge_weights.reshape(-1), (0, pad_e))
    ep = jnp.pad(edge_params.reshape(-1), (0, pad_e))

    out = _sc_msg(x, coords, ew, ep)
    return out[:NODES].reshape(B, N, D_OUT)


# speedup vs baseline: 1.9081x; 1.1365x over previous
import functools

import jax
import jax.numpy as jnp
from jax import lax
from jax.experimental import pallas as pl
from jax.experimental.pallas import tpu as pltpu
from jax.experimental.pallas import tpu_sc as plsc

B, N, K = 2, 5000, 16
D_IN, D_COND, D_OUT = 256, 128, 256
NODES = B * N
NW = 32
C = 320
PAD_NODES = NW * C
SUB = 8
NSUB = C // SUB
ROWS_BLK = 2000
LANES = 16


def _dense_body(nf_ref, cond_ref, wc_ref, bc_ref, wf_ref, bf_ref, x_ref):
    j = pl.program_id(0)
    x = jnp.dot(nf_ref[...], wf_ref[...], preferred_element_type=jnp.float32)
    x = x + bf_ref[...]
    mu = jnp.mean(x, axis=-1, keepdims=True)
    var = jnp.mean((x - mu) ** 2, axis=-1, keepdims=True)
    x = (x - mu) * lax.rsqrt(var + 1e-5)
    gb = jnp.dot(cond_ref[...], wc_ref[...], preferred_element_type=jnp.float32)
    gb = gb + bc_ref[...]
    gamma = gb[:, :D_OUT] + 1.0
    beta = gb[:, D_OUT:]
    rows = j * ROWS_BLK + lax.broadcasted_iota(jnp.int32, (ROWS_BLK, 1), 0)
    in_b1 = rows >= N
    g = jnp.where(in_b1, gamma[1:2, :], gamma[0:1, :])
    bt = jnp.where(in_b1, beta[1:2, :], beta[0:1, :])
    x = jnp.maximum(x * g + bt, 0.0)
    u = lax.bitcast_convert_type(x, jnp.int32)
    r = lax.shift_right_logical(
        u + 0x7FFF + jnp.bitwise_and(lax.shift_right_logical(u, 16), 1), 16)
    x_ref[...] = jnp.bitwise_or(
        r[:, : D_OUT // 2], lax.shift_left(r[:, D_OUT // 2:], 16))


def _dense(nf, cond2d, Wc, bc2d, Wf, bf2d):
    return pl.pallas_call(
        _dense_body,
        grid=(NODES // ROWS_BLK,),
        in_specs=[
            pl.BlockSpec((ROWS_BLK, D_IN), lambda j: (j, 0)),
            pl.BlockSpec((B, D_COND), lambda j: (0, 0)),
            pl.BlockSpec((D_COND, 2 * D_OUT), lambda j: (0, 0)),
            pl.BlockSpec((1, 2 * D_OUT), lambda j: (0, 0)),
            pl.BlockSpec((D_IN, D_OUT), lambda j: (0, 0)),
            pl.BlockSpec((1, D_OUT), lambda j: (0, 0)),
        ],
        out_specs=pl.BlockSpec((ROWS_BLK, D_OUT // 2), lambda j: (j, 0)),
        out_shape=jax.ShapeDtypeStruct((NODES, D_OUT // 2), jnp.int32),
    )(nf, cond2d, Wc, bc2d, Wf, bf2d)


def _unpack2(w):
    a = lax.bitcast_convert_type(lax.shift_left(w, 16), jnp.float32)
    b = lax.bitcast_convert_type(jnp.bitwise_and(w, jnp.int32(-65536)), jnp.float32)
    return a, b


def _pack2(a, b):
    ua = lax.bitcast_convert_type(a, jnp.int32)
    ub = lax.bitcast_convert_type(b, jnp.int32)
    ra = lax.shift_right_logical(
        ua + 0x7FFF + jnp.bitwise_and(lax.shift_right_logical(ua, 16), 1), 16)
    rb = ub + 0x7FFF + jnp.bitwise_and(lax.shift_right_logical(ub, 16), 1)
    return jnp.bitwise_or(ra, jnp.bitwise_and(rb, jnp.int32(-65536)))


def _sc_body(x_hbm, coords_hbm, ew_hbm, ep_hbm, out_hbm,
             coords_v, ew_v, ep_v, xres0, xres1, out_v,
             rows0, rows1, semr0, semr1, semx0, semx1):
    cid = lax.axis_index("c")
    sid = lax.axis_index("s")
    wid = sid * 2 + cid
    base_node = wid * C
    base_edge = base_node * K

    pltpu.sync_copy(coords_hbm.at[pl.ds(base_edge, C * K)], coords_v)
    pltpu.sync_copy(ew_hbm.at[pl.ds(base_edge, C * K)], ew_v)
    pltpu.sync_copy(ep_hbm.at[pl.ds(base_edge, C * K)], ep_v)

    rows = (rows0, rows1)
    xres = (xres0, xres1)
    semr = (semr0, semr1)
    semx = (semx0, semx1)

    def issue(s, b):
        e0 = s * SUB * K
        pltpu.async_copy(
            x_hbm.at[coords_v.at[pl.ds(e0, SUB * K)]], rows[b], semr[b])
        node0 = base_node + s * SUB
        row0 = jnp.minimum(node0, NODES - SUB)
        pltpu.async_copy(x_hbm.at[pl.ds(row0, SUB)], xres[b], semx[b])

    def drain(b):
        pltpu.make_async_copy(x_hbm.at[pl.ds(0, SUB * K)], rows[b], semr[b]).wait()

    def compute(s, b):
        nch = D_OUT // (2 * LANES)

        def node_body(i, carry):
            e0 = (s * SUB + i) * K
            wv = ew_v[pl.ds(e0, LANES)] * ep_v[pl.ds(e0, LANES)]
            acca = [jnp.zeros((LANES,), jnp.float32)] * nch
            accb = [jnp.zeros((LANES,), jnp.float32)] * nch
            for k in range(K):
                wk = wv.at[jnp.full((LANES,), k, jnp.int32)].get(
                    mode="promise_in_bounds")
                r = i * K + k
                for d in range(nch):
                    va, vb = _unpack2(rows[b][r, pl.ds(d * LANES, LANES)])
                    acca[d] = acca[d] + wk * va
                    accb[d] = accb[d] + wk * vb
            for d in range(nch):
                xa, xb = _unpack2(xres[b][i, pl.ds(d * LANES, LANES)])
                ra = jnp.maximum(acca[d] + xa, 0.0)
                rb = jnp.maximum(accb[d] + xb, 0.0)
                out_v[i, pl.ds(d * LANES, LANES)] = _pack2(ra, rb)
            return carry

        pltpu.make_async_copy(x_hbm.at[pl.ds(0, SUB)], xres[b], semx[b]).wait()
        lax.fori_loop(0, SUB, node_body, 0)
        pltpu.sync_copy(out_v, out_hbm.at[pl.ds(base_node + s * SUB, SUB)])

    issue(0, 0)

    def outer(g, carry):
        s0 = 2 * g
        s1 = 2 * g + 1
        issue(s1, 1)
        drain(0)
        compute(s0, 0)
        issue(jnp.minimum(s1 + 1, NSUB - 1), 0)
        drain(1)
        compute(s1, 1)
        return carry

    lax.fori_loop(0, NSUB // 2, outer, 0)
    drain(0)
    pltpu.make_async_copy(x_hbm.at[pl.ds(0, SUB)], xres[0], semx[0]).wait()


_sc_msg = functools.partial(
    pl.kernel,
    mesh=plsc.VectorSubcoreMesh(core_axis_name="c", subcore_axis_name="s"),
    out_type=jax.ShapeDtypeStruct((PAD_NODES, D_OUT // 2), jnp.int32),
    scratch_types=[
        pltpu.VMEM((C * K,), jnp.int32),
        pltpu.VMEM((C * K,), jnp.float32),
        pltpu.VMEM((C * K,), jnp.float32),
        pltpu.VMEM((SUB, D_OUT // 2), jnp.int32),
        pltpu.VMEM((SUB, D_OUT // 2), jnp.int32),
        pltpu.VMEM((SUB, D_OUT // 2), jnp.int32),
        pltpu.VMEM((SUB * K, D_OUT // 2), jnp.int32),
        pltpu.VMEM((SUB * K, D_OUT // 2), jnp.int32),
        pltpu.SemaphoreType.DMA,
        pltpu.SemaphoreType.DMA,
        pltpu.SemaphoreType.DMA,
        pltpu.SemaphoreType.DMA,
    ],
)(_sc_body)


def kernel(node_feats, cond_feats, edge_weights, edge_params, coords1, Wc, bc, Wf, bf):
    nf = node_feats.reshape(NODES, D_IN)
    cond2d = cond_feats.reshape(B, D_COND)
    x = _dense(nf, cond2d, Wc, bc.reshape(1, -1), Wf, bf.reshape(1, -1))

    pad_e = (PAD_NODES - NODES) * K
    coords = jnp.pad(coords1.astype(jnp.int32), (0, pad_e))
    ew = jnp.pad(edge_weights.reshape(-1), (0, pad_e))
    ep = jnp.pad(edge_params.reshape(-1), (0, pad_e))

    out = _sc_msg(x, coords, ew, ep)[:NODES]
    lo = lax.bitcast_convert_type(lax.shift_left(out, 16), jnp.float32)
    hi = lax.bitcast_convert_type(
        jnp.bitwise_and(out, jnp.int32(-65536)), jnp.float32)
    return jnp.concatenate([lo, hi], axis=-1).reshape(B, N, D_OUT)
